# baseline (speedup 1.0000x reference)
import functools

import jax
import jax.numpy as jnp
from jax import lax
from jax.experimental import pallas as pl
from jax.experimental.pallas import tpu as pltpu
from jax.experimental.pallas import tpu_sc as plsc

D_MODEL = 64
BATCH = 16384
_ROWS_PER_TILE = 8

_NC = 2
_NS = 16
_NW = _NC * _NS
_B_PER_W = BATCH // _NW

_mesh = plsc.VectorSubcoreMesh(core_axis_name="c", subcore_axis_name="s")


@functools.partial(
    pl.kernel,
    mesh=_mesh,
    out_type=jax.ShapeDtypeStruct((BATCH, D_MODEL), jnp.float32),
    scratch_types=[
        pltpu.VMEM((_B_PER_W,), jnp.int32),
        pltpu.VMEM((_B_PER_W, D_MODEL), jnp.float32),
        pltpu.SemaphoreType.DMA,
    ],
    compiler_params=pltpu.CompilerParams(has_side_effects=True),
)
def _gather_kernel(idx_hbm, table_hbm, out_hbm, idx_v, rows_v, sem):
    wid = lax.axis_index("s") * _NC + lax.axis_index("c")
    base = wid * _B_PER_W
    pltpu.sync_copy(idx_hbm.at[pl.ds(base, _B_PER_W)], idx_v)

    def fire(g, _):
        ivec = idx_v[pl.ds(g * 16, 16)]
        for l in range(16):
            i0 = ivec[l]
            pltpu.make_async_copy(
                table_hbm.at[i0],
                rows_v.at[g * 16 + l],
                sem,
            ).start()
        return 0

    lax.fori_loop(0, _B_PER_W // 16, fire, 0)

    def drain(i, _):
        pltpu.make_async_copy(
            table_hbm.at[0], rows_v.at[0], sem
        ).wait()
        return 0

    lax.fori_loop(0, _B_PER_W, drain, 0)

    pltpu.sync_copy(rows_v, out_hbm.at[pl.ds(base, _B_PER_W)])


def kernel(voice_ids, embedding_table):
    return _gather_kernel(voice_ids.astype(jnp.int32), embedding_table)


# speedup vs baseline: 1.0139x; 1.0139x over previous
import functools

import jax
import jax.numpy as jnp
from jax import lax
from jax.experimental import pallas as pl
from jax.experimental.pallas import tpu as pltpu
from jax.experimental.pallas import tpu_sc as plsc

D_MODEL = 64
BATCH = 16384
NUM_VOICES = 1000000

_NC = 2
_NS = 16
_NW = _NC * _NS
_NTILES = NUM_VOICES // 128
_TAIL0 = _NTILES * 128
_BASE_T = _NTILES // _NW
_EXTRA = _NTILES - _BASE_T * _NW
_CAP = 768
_NRING = 8
_NSTAGE = 16

_mesh = plsc.VectorSubcoreMesh(core_axis_name="c", subcore_axis_name="s")


@functools.partial(
    pl.kernel,
    mesh=_mesh,
    out_type=jax.ShapeDtypeStruct((BATCH + 8, 128), jnp.float32),
    scratch_types=[
        pltpu.VMEM((BATCH,), jnp.int32),
        pltpu.VMEM((_CAP + 32,), jnp.int32),
        pltpu.VMEM((_CAP + 32,), jnp.int32),
        pltpu.VMEM((_CAP + 32,), jnp.int32),
        pltpu.VMEM((_CAP + 32,), jnp.int32),
        pltpu.VMEM((_NRING, D_MODEL, 128), jnp.float32),
        pltpu.VMEM((_NSTAGE, 128), jnp.float32),
        pltpu.SMEM((256,), jnp.int32),
        pltpu.SMEM((256,), jnp.int32),
        pltpu.SMEM((256,), jnp.int32),
        pltpu.SMEM((256,), jnp.int32),
        pltpu.SMEM((_NSTAGE,), jnp.int32),
        [pltpu.SemaphoreType.DMA for _ in range(_NRING)],
        [pltpu.SemaphoreType.DMA for _ in range(_NSTAGE)],
    ],
    compiler_params=pltpu.CompilerParams(needs_layout_passes=False),
)
def _gather_kernel(idx_hbm, tableT_hbm, tail_hbm, outP_hbm, idx_v, vs_u,
                   bs_u, vs_s, bs_s, ring_v, stage_v, hist_s, starts_s,
                   cur_s, tlist_s, pend_s, sems_t, sems_o):
    w = lax.axis_index("s") * _NC + lax.axis_index("c")
    tstart = _BASE_T * w + jnp.minimum(w, _EXTRA)
    ntiles = jnp.where(w < _EXTRA, _BASE_T + 1, _BASE_T)
    lo = tstart * 128
    hi = (tstart + ntiles) * 128
    iota = lax.iota(jnp.int32, 16)
    zero16 = jnp.zeros((16,), jnp.int32)

    pltpu.sync_copy(idx_hbm, idx_v)

    def scan(g, cnt):
        v = idx_v[pl.ds(g * 16, 16)]
        b = iota + g * 16
        m = (v >= lo) & (v < hi)
        plsc.store_compressed(vs_u.at[pl.ds(cnt, 16)], v, mask=m)
        plsc.store_compressed(bs_u.at[pl.ds(cnt, 16)], b, mask=m)
        n = plsc.all_reduce_population_count(m)[0]
        return jnp.minimum(cnt + n, _CAP)

    cnt = lax.fori_loop(0, BATCH // 16, scan, 0)

    def zinit(t, _):
        hist_s[t] = 0
        return 0

    lax.fori_loop(0, 256, zinit, 0)
    for l in range(_NSTAGE):
        pend_s[l] = 0

    def histloop(j, _):
        tv = vs_u[pl.ds(j * 16, 16)]
        tid_v = jnp.where(iota + j * 16 < cnt,
                          jnp.clip((tv >> 7) - tstart, 0, 254), 255)
        for l in range(16):
            t_l = tid_v[l]
            hist_s[t_l] = hist_s[t_l] + 1
        return 0

    lax.fori_loop(0, (_CAP + 15) // 16, histloop, 0)

    def prefix(t, carry):
        acc, m = carry
        starts_s[t] = acc
        cur_s[t] = acc
        h = hist_s[t]

        @pl.when(h > 0)
        def _():
            tlist_s[m] = t

        return acc + h, m + jnp.where(h > 0, 1, 0)

    acc, ntouched = lax.fori_loop(0, _BASE_T + 1, prefix, (0, 0))
    starts_s[_BASE_T + 1] = acc

    def place(j, _):
        tv = vs_u[pl.ds(j * 16, 16)]
        bv = bs_u[pl.ds(j * 16, 16)]
        tid_v = jnp.where(iota + j * 16 < cnt,
                          jnp.clip((tv >> 7) - tstart, 0, 254), 255)
        posv = zero16
        for l in range(16):
            t_l = tid_v[l]
            p_l = cur_s[t_l]
            cur_s[t_l] = p_l + 1
            posv = jnp.where(iota == l, p_l, posv)
        posv = jnp.clip(posv, 0, _CAP + 31)
        mvalid = (iota + j * 16) < cnt
        plsc.store_scatter(vs_s, [posv], tv, mask=mvalid)
        plsc.store_scatter(bs_s, [posv], bv, mask=mvalid)
        return 0

    lax.fori_loop(0, (_CAP + 15) // 16, place, 0)

    def fire_tile(i, slot):
        t = jnp.clip(tlist_s[i], 0, _BASE_T)
        off = pl.multiple_of(jnp.clip((tstart + t) * 128, 0, _TAIL0 - 128),
                             128)
        pltpu.make_async_copy(
            tableT_hbm.at[:, pl.ds(off, 128)], ring_v.at[slot], sems_t[slot]
        ).start()

    for r in range(_NRING):
        @pl.when(r < ntouched)
        def _():
            fire_tile(r, r)

    def process_tile(i, slot):
        pltpu.make_async_copy(
            tableT_hbm.at[:, pl.ds(0, 128)], ring_v.at[slot], sems_t[slot]
        ).wait()
        t = jnp.clip(tlist_s[i], 0, _BASE_T)
        lo_t = starts_s[t]
        hi_t = starts_s[t + 1]
        slotv16 = zero16 + slot

        def group_loop(g, _):
            vv = vs_s[pl.ds(g * 16, 16)]
            bv2 = bs_s[pl.ds(g * 16, 16)]
            for l in range(16):
                pos = g * 16 + l

                @pl.when((pos >= lo_t) & (pos < hi_t))
                def _():
                    v0 = vv[l]
                    b0 = jnp.clip(bv2[l], 0, BATCH - 1)
                    col = zero16 + (v0 & 127)

                    @pl.when(pend_s[l] == 1)
                    def _():
                        pltpu.make_async_copy(
                            outP_hbm.at[BATCH], stage_v.at[l], sems_o[l]
                        ).wait()

                    sv16 = zero16 + l
                    for k in range(4):
                        vals = plsc.load_gather(
                            ring_v, [slotv16, iota + k * 16, col])
                        plsc.store_scatter(
                            stage_v, [sv16, iota + k * 16], vals)
                    pltpu.make_async_copy(
                        stage_v.at[l], outP_hbm.at[b0], sems_o[l]
                    ).start()
                    pend_s[l] = 1
            return 0

        lax.fori_loop(lo_t >> 4, (hi_t + 15) >> 4, group_loop, 0)

        @pl.when(i + _NRING < ntouched)
        def _():
            fire_tile(i + _NRING, slot)

    def round_loop(rnd, _):
        for r in range(_NRING):
            i = rnd * _NRING + r

            @pl.when(i < ntouched)
            def _():
                process_tile(i, r)
        return 0

    lax.fori_loop(0, (ntouched + _NRING - 1) // _NRING, round_loop, 0)

    for l in range(_NSTAGE):
        @pl.when(pend_s[l] == 1)
        def _():
            pltpu.make_async_copy(
                outP_hbm.at[BATCH], stage_v.at[l], sems_o[l]
            ).wait()

    @pl.when(w == _NW - 1)
    def _():
        def tail_scan(g, _):
            v = idx_v[pl.ds(g * 16, 16)]
            for l in range(16):
                @pl.when(v[l] >= _TAIL0)
                def _():
                    row = jnp.clip(v[l] - _TAIL0, 0, NUM_VOICES - _TAIL0 - 1)
                    b0 = g * 16 + l
                    pltpu.sync_copy(tail_hbm.at[row], stage_v.at[0])
                    pltpu.sync_copy(stage_v.at[0], outP_hbm.at[b0])
            return 0

        lax.fori_loop(0, BATCH // 16, tail_scan, 0)


def kernel(voice_ids, embedding_table):
    tail = jnp.pad(embedding_table[_TAIL0:, :], ((0, 0), (0, 128 - D_MODEL)))
    outP = _gather_kernel(voice_ids.astype(jnp.int32), embedding_table.T,
                          tail)
    return outP[:BATCH, :D_MODEL]


# speedup vs baseline: 1.0425x; 1.0283x over previous
import functools

import jax
import jax.numpy as jnp
from jax import lax
from jax.experimental import pallas as pl
from jax.experimental.pallas import tpu as pltpu
from jax.experimental.pallas import tpu_sc as plsc

D_MODEL = 64
BATCH = 16384
NUM_VOICES = 1000000

_NC = 2
_NS = 16
_NW = _NC * _NS
_NTILES = NUM_VOICES // 128
_TAIL0 = _NTILES * 128
_BASE_T = _NTILES // _NW
_EXTRA = _NTILES - _BASE_T * _NW
_CAP = 768
_NRING = 8
_NSTAGE = 16

_mesh = plsc.VectorSubcoreMesh(core_axis_name="c", subcore_axis_name="s")


@functools.partial(
    pl.kernel,
    mesh=_mesh,
    out_type=jax.ShapeDtypeStruct((BATCH + 8, 128), jnp.float32),
    scratch_types=[
        pltpu.VMEM((BATCH,), jnp.int32),
        pltpu.VMEM((_CAP + 32,), jnp.int32),
        pltpu.VMEM((_CAP + 32,), jnp.int32),
        pltpu.VMEM((_CAP + 32,), jnp.int32),
        pltpu.VMEM((_CAP + 32,), jnp.int32),
        pltpu.VMEM((_NRING, D_MODEL, 128), jnp.float32),
        pltpu.VMEM((_NSTAGE, 128), jnp.float32),
        pltpu.SMEM((256,), jnp.int32),
        pltpu.SMEM((256,), jnp.int32),
        pltpu.SMEM((256,), jnp.int32),
        pltpu.SMEM((256,), jnp.int32),
        pltpu.SMEM((_NSTAGE,), jnp.int32),
        [pltpu.SemaphoreType.DMA for _ in range(_NRING)],
        [pltpu.SemaphoreType.DMA for _ in range(_NSTAGE)],
    ],
    compiler_params=pltpu.CompilerParams(needs_layout_passes=False),
)
def _gather_kernel(idx_hbm, tableT_hbm, tail_hbm, outP_hbm, idx_v, vs_u,
                   bs_u, vs_s, bs_s, ring_v, stage_v, hist_s, starts_s,
                   cur_s, tlist_s, pend_s, sems_t, sems_o):
    w = lax.axis_index("s") * _NC + lax.axis_index("c")
    tstart = _BASE_T * w + jnp.minimum(w, _EXTRA)
    ntiles = jnp.where(w < _EXTRA, _BASE_T + 1, _BASE_T)
    lo = tstart * 128
    hi = (tstart + ntiles) * 128
    iota = lax.iota(jnp.int32, 16)
    zero16 = jnp.zeros((16,), jnp.int32)

    pltpu.sync_copy(idx_hbm, idx_v)

    def scan(g, cnt):
        for u in range(2):
            v = idx_v[pl.ds((g * 2 + u) * 16, 16)]
            b = iota + (g * 2 + u) * 16
            m = (v >= lo) & (v < hi)
            plsc.store_compressed(vs_u.at[pl.ds(cnt, 16)], v, mask=m)
            plsc.store_compressed(bs_u.at[pl.ds(cnt, 16)], b, mask=m)
            n = plsc.all_reduce_population_count(m)[0]
            cnt = jnp.minimum(cnt + n, _CAP)
        return cnt

    cnt = lax.fori_loop(0, BATCH // 32, scan, 0)

    def zinit(t, _):
        hist_s[t] = 0
        return 0

    lax.fori_loop(0, 256, zinit, 0)
    for l in range(_NSTAGE):
        pend_s[l] = 0

    def histloop(j, _):
        tv = vs_u[pl.ds(j * 16, 16)]
        tid_v = jnp.where(iota + j * 16 < cnt,
                          jnp.clip((tv >> 7) - tstart, 0, 254), 255)
        for l in range(16):
            t_l = tid_v[l]
            hist_s[t_l] = hist_s[t_l] + 1
        return 0

    lax.fori_loop(0, (_CAP + 15) // 16, histloop, 0)

    def prefix(t, carry):
        acc, m = carry
        starts_s[t] = acc
        cur_s[t] = acc
        h = hist_s[t]

        @pl.when(h > 0)
        def _():
            tlist_s[m] = t

        return acc + h, m + jnp.where(h > 0, 1, 0)

    acc, ntouched = lax.fori_loop(0, _BASE_T + 1, prefix, (0, 0))
    starts_s[_BASE_T + 1] = acc

    def place(j, _):
        tv = vs_u[pl.ds(j * 16, 16)]
        bv = bs_u[pl.ds(j * 16, 16)]
        tid_v = jnp.where(iota + j * 16 < cnt,
                          jnp.clip((tv >> 7) - tstart, 0, 254), 255)
        posv = zero16
        for l in range(16):
            t_l = tid_v[l]
            p_l = cur_s[t_l]
            cur_s[t_l] = p_l + 1
            posv = jnp.where(iota == l, p_l, posv)
        posv = jnp.clip(posv, 0, _CAP + 31)
        mvalid = (iota + j * 16) < cnt
        plsc.store_scatter(vs_s, [posv], tv, mask=mvalid)
        plsc.store_scatter(bs_s, [posv], bv, mask=mvalid)
        return 0

    lax.fori_loop(0, (_CAP + 15) // 16, place, 0)

    def fire_tile(i, slot):
        t = jnp.clip(tlist_s[i], 0, _BASE_T)
        off = pl.multiple_of(jnp.clip((tstart + t) * 128, 0, _TAIL0 - 128),
                             128)
        for g in range(8):
            pltpu.make_async_copy(
                tableT_hbm.at[pl.ds(g * 8, 8), pl.ds(off, 128)],
                ring_v.at[slot].at[pl.ds(g * 8, 8)],
                sems_t[slot],
            ).start()

    for r in range(_NRING):
        @pl.when(r < ntouched)
        def _():
            fire_tile(r, r)

    def process_tile(i, slot):
        for g in range(8):
            pltpu.make_async_copy(
                tableT_hbm.at[pl.ds(0, 8), pl.ds(0, 128)],
                ring_v.at[slot].at[pl.ds(0, 8)],
                sems_t[slot],
            ).wait()
        t = jnp.clip(tlist_s[i], 0, _BASE_T)
        lo_t = starts_s[t]
        hi_t = starts_s[t + 1]
        slotv16 = zero16 + slot

        def group_loop(g, _):
            vv = vs_s[pl.ds(g * 16, 16)]
            bv2 = bs_s[pl.ds(g * 16, 16)]
            for l in range(16):
                pos = g * 16 + l

                @pl.when((pos >= lo_t) & (pos < hi_t))
                def _():
                    v0 = vv[l]
                    b0 = jnp.clip(bv2[l], 0, BATCH - 1)
                    col = zero16 + (v0 & 127)

                    @pl.when(pend_s[l] == 1)
                    def _():
                        pltpu.make_async_copy(
                            outP_hbm.at[BATCH], stage_v.at[l], sems_o[l]
                        ).wait()

                    sv16 = zero16 + l
                    for k in range(4):
                        vals = plsc.load_gather(
                            ring_v, [slotv16, iota + k * 16, col])
                        plsc.store_scatter(
                            stage_v, [sv16, iota + k * 16], vals)
                    pltpu.make_async_copy(
                        stage_v.at[l], outP_hbm.at[b0], sems_o[l]
                    ).start()
                    pend_s[l] = 1
            return 0

        lax.fori_loop(lo_t >> 4, (hi_t + 15) >> 4, group_loop, 0)

        @pl.when(i + _NRING < ntouched)
        def _():
            fire_tile(i + _NRING, slot)

    def round_loop(rnd, _):
        for r in range(_NRING):
            i = rnd * _NRING + r

            @pl.when(i < ntouched)
            def _():
                process_tile(i, r)
        return 0

    lax.fori_loop(0, (ntouched + _NRING - 1) // _NRING, round_loop, 0)

    for l in range(_NSTAGE):
        @pl.when(pend_s[l] == 1)
        def _():
            pltpu.make_async_copy(
                outP_hbm.at[BATCH], stage_v.at[l], sems_o[l]
            ).wait()

    def tail_scan(g, _):
        gg = w * (BATCH // _NW // 16) + g
        v = idx_v[pl.ds(gg * 16, 16)]
        for l in range(16):
            @pl.when(v[l] >= _TAIL0)
            def _():
                row = jnp.clip(v[l] - _TAIL0, 0, NUM_VOICES - _TAIL0 - 1)
                b0 = gg * 16 + l
                pltpu.sync_copy(tail_hbm.at[row], stage_v.at[0])
                pltpu.sync_copy(stage_v.at[0], outP_hbm.at[b0])
        return 0

    lax.fori_loop(0, BATCH // _NW // 16, tail_scan, 0)


def kernel(voice_ids, embedding_table):
    tail = jnp.pad(embedding_table[_TAIL0:, :], ((0, 0), (0, 128 - D_MODEL)))
    outP = _gather_kernel(voice_ids.astype(jnp.int32), embedding_table.T,
                          tail)
    return outP[:BATCH, :D_MODEL]


# speedup vs baseline: 1.4842x; 1.4236x over previous
import functools

import jax
import jax.numpy as jnp
from jax import lax
from jax.experimental import pallas as pl
from jax.experimental.pallas import tpu as pltpu
from jax.experimental.pallas import tpu_sc as plsc

D_MODEL = 64
BATCH = 16384
_ROWS_PER_TILE = 8

_NC = 2
_NS = 16
_NW = _NC * _NS
_B_PER_W = BATCH // _NW

_mesh = plsc.VectorSubcoreMesh(core_axis_name="c", subcore_axis_name="s")


@functools.partial(
    pl.kernel,
    mesh=_mesh,
    out_type=jax.ShapeDtypeStruct((BATCH, D_MODEL), jnp.float32),
    scratch_types=[
        pltpu.VMEM((_B_PER_W,), jnp.int32),
        pltpu.VMEM((_B_PER_W, D_MODEL), jnp.float32),
        pltpu.SemaphoreType.DMA,
    ],
)
def _gather_kernel(idx_hbm, table_hbm, out_hbm, idx_v, rows_v, sem):
    wid = lax.axis_index("s") * _NC + lax.axis_index("c")
    base = wid * _B_PER_W
    pltpu.sync_copy(idx_hbm.at[pl.ds(base, _B_PER_W)], idx_v)

    def fire(g, _):
        ivec = idx_v[pl.ds(g * 16, 16)]
        for l in range(16):
            i0 = ivec[l]
            pltpu.make_async_copy(
                table_hbm.at[i0 >> 3, i0 & 7],
                rows_v.at[g * 16 + l],
                sem,
            ).start()
        return 0

    lax.fori_loop(0, _B_PER_W // 16, fire, 0)

    def drain(i, _):
        pltpu.make_async_copy(
            table_hbm.at[0, 0], rows_v.at[0], sem
        ).wait()
        return 0

    lax.fori_loop(0, _B_PER_W, drain, 0)

    pltpu.sync_copy(rows_v, out_hbm.at[pl.ds(base, _B_PER_W)])


def kernel(voice_ids, embedding_table):
    num_voices = embedding_table.shape[0]
    table3 = embedding_table.reshape(num_voices // _ROWS_PER_TILE,
                                     _ROWS_PER_TILE, D_MODEL)
    return _gather_kernel(voice_ids.astype(jnp.int32), table3)
